# Initial kernel scaffold; baseline (speedup 1.0000x reference)
import jax
import jax.numpy as jnp
from jax.experimental import pallas as pl


def kernel(x, wq, wk, wv, wo):
    raise NotImplementedError("write your pallas kernel here")


# speedup vs baseline: 73.3339x; 73.3339x over previous
import jax
import jax.numpy as jnp
from jax.experimental import pallas as pl
from jax.experimental.pallas import tpu as pltpu

_B, _T, _DIMS, _NHEADS = 1, 2048, 768, 12
_HD = _DIMS // _NHEADS
_TAU = 0.1

_RP = 256
_R = 256
_PRE = 8
_M = _R + _PRE


def _proj_kernel(x_ref, wq_ref, wk_ref, wv_ref, q_ref, k_ref, v_ref):
    x = x_ref[...]
    for w_ref, o_ref in ((wq_ref, q_ref), (wk_ref, k_ref), (wv_ref, v_ref)):
        y = jax.lax.dot_general(x, w_ref[...], (((1,), (1,)), ((), ())),
                                preferred_element_type=jnp.float32)
        y = y * (1.0 / _TAU)
        y = y - jnp.max(y, axis=1, keepdims=True)
        e = jnp.exp(y)
        o_ref[0] = e / jnp.sum(e, axis=1, keepdims=True)


def _shift_diag(x, s, fill):
    m, t = x.shape
    x = jnp.concatenate([jnp.full((s, t), fill, x.dtype), x[:m - s, :]], axis=0)
    x = jnp.concatenate([jnp.full((m, s), fill, x.dtype), x[:, :t - s]], axis=1)
    return x


def _attn_kernel(q_ref, k_ref, v_ref, wo_ref, o_ref, carry_ref):
    rb = pl.program_id(1)
    q = q_ref[0]
    k = k_ref[0]
    a = jax.lax.dot_general(q, k, (((1,), (1,)), ((), ())),
                            preferred_element_type=jnp.float32)

    carry = jnp.where(rb == 0, 0.0, carry_ref[...])
    carry = jnp.concatenate(
        [carry[:, _PRE - 1:], jnp.zeros((1, _PRE - 1), jnp.float32)], axis=1)

    g = jnp.concatenate(
        [jnp.zeros((1, _T), jnp.float32),
         jnp.ones((_PRE - 1, _T), jnp.float32), a], axis=0)
    b = jnp.concatenate(
        [carry, jnp.zeros((_PRE - 1, _T), jnp.float32), a], axis=0)

    s = 1
    while s < _M:
        gs = _shift_diag(g, s, 1.0)
        bs = _shift_diag(b, s, 0.0)
        b = g * bs + b
        g = g * gs
        s *= 2

    carry_ref[...] = b[_M - 1:_M, :]
    y = b[_PRE:, :]

    rows = rb * _R + jax.lax.broadcasted_iota(jnp.int32, (_R, _T), 0)
    cols = jax.lax.broadcasted_iota(jnp.int32, (_R, _T), 1)
    p = cols.astype(jnp.float32) / (rows.astype(jnp.float32) + 1.0)
    z = jnp.where(cols <= rows, (y + p) * (1.0 / _TAU), -jnp.inf)
    z = z - jnp.max(z, axis=1, keepdims=True)
    e = jnp.exp(z)
    probs = e / jnp.sum(e, axis=1, keepdims=True)

    av = jnp.dot(probs, v_ref[0], preferred_element_type=jnp.float32)
    o_ref[0] = jnp.dot(av, wo_ref[0], preferred_element_type=jnp.float32)


def kernel(x, wq, wk, wv, wo):
    x2 = x.reshape(_T, _DIMS)

    q_sm, k_sm, v_sm = pl.pallas_call(
        _proj_kernel,
        grid=(_T // _RP, _NHEADS),
        in_specs=[
            pl.BlockSpec((_RP, _DIMS), lambda rb, h: (rb, 0)),
            pl.BlockSpec((_HD, _DIMS), lambda rb, h: (h, 0)),
            pl.BlockSpec((_HD, _DIMS), lambda rb, h: (h, 0)),
            pl.BlockSpec((_HD, _DIMS), lambda rb, h: (h, 0)),
        ],
        out_specs=[
            pl.BlockSpec((1, _RP, _HD), lambda rb, h: (h, rb, 0)),
            pl.BlockSpec((1, _RP, _HD), lambda rb, h: (h, rb, 0)),
            pl.BlockSpec((1, _RP, _HD), lambda rb, h: (h, rb, 0)),
        ],
        out_shape=[jax.ShapeDtypeStruct((_NHEADS, _T, _HD), jnp.float32)] * 3,
    )(x2, wq, wk, wv)

    out = pl.pallas_call(
        _attn_kernel,
        grid=(_NHEADS, _T // _R),
        in_specs=[
            pl.BlockSpec((1, _R, _HD), lambda h, rb: (h, rb, 0)),
            pl.BlockSpec((1, _T, _HD), lambda h, rb: (h, 0, 0)),
            pl.BlockSpec((1, _T, _HD), lambda h, rb: (h, 0, 0)),
            pl.BlockSpec((1, _HD, _HD), lambda h, rb: (h, 0, 0)),
        ],
        out_specs=pl.BlockSpec((1, _R, _HD), lambda h, rb: (h, rb, 0)),
        out_shape=jax.ShapeDtypeStruct((_NHEADS, _T, _HD), jnp.float32),
        scratch_shapes=[pltpu.VMEM((1, _T), jnp.float32)],
    )(q_sm, k_sm, v_sm, wo.reshape(_NHEADS, _HD, _HD))

    return out.transpose(1, 0, 2).reshape(_B, _T, _DIMS)


# speedup vs baseline: 103.4842x; 1.4111x over previous
import jax
import jax.numpy as jnp
import numpy as np
from jax.experimental import pallas as pl
from jax.experimental.pallas import tpu as pltpu

_B, _T, _DIMS, _NHEADS = 1, 2048, 768, 12
_HD = _DIMS // _NHEADS
_TAU = 0.1

_RP = 256
_R = 256
_PRE = 8
_M = _R + _PRE
_CH = 8
_NC = _M // _CH


def _proj_kernel(x_ref, wq_ref, wk_ref, wv_ref, q_ref, k_ref, v_ref):
    x = x_ref[...]
    for w_ref, o_ref in ((wq_ref, q_ref), (wk_ref, k_ref), (wv_ref, v_ref)):
        y = jax.lax.dot_general(x, w_ref[...], (((1,), (1,)), ((), ())),
                                preferred_element_type=jnp.float32)
        y = y * (1.0 / _TAU)
        y = y - jnp.max(y, axis=1, keepdims=True)
        e = jnp.exp(y)
        o_ref[0] = e / jnp.sum(e, axis=1, keepdims=True)


def _shift_diag(x, s, fill):
    m, t = x.shape
    x = jnp.concatenate([jnp.full((s, t), fill, x.dtype), x[:m - s, :]], axis=0)
    x = jnp.concatenate([jnp.full((m, s), fill, x.dtype), x[:, :t - s]], axis=1)
    return x


def _shift_cols(x, s, fill):
    sh = x.shape[:-1] + (s,)
    return jnp.concatenate(
        [jnp.full(sh, fill, x.dtype), x[..., :x.shape[-1] - s]], axis=-1)


_NG = (_NC + _CH - 1) // _CH


def _scan_stripe(g, b):
    g3 = g.reshape(_NC, _CH, _T)
    b3 = b.reshape(_NC, _CH, _T)
    rnp = jax.lax.broadcasted_iota(jnp.int32, (1, _CH, _T), 1)
    cnp = jax.lax.broadcasted_iota(jnp.int32, (1, _CH, _T), 2)

    for s in (1, 2, 4):
        fill = (rnp < s) | (cnp < s)
        gs = jnp.where(fill, 1.0, pltpu.roll(pltpu.roll(g3, s, 1), s, 2))
        bs = jnp.where(fill, 0.0, pltpu.roll(pltpu.roll(b3, s, 1), s, 2))
        b3 = g3 * bs + b3
        g3 = g3 * gs

    pad = _NG * _CH - _NC
    gc = jnp.concatenate(
        [g3[:, _CH - 1, :], jnp.ones((pad, _T), jnp.float32)],
        axis=0).reshape(_NG, _CH, _T)
    bc = jnp.concatenate(
        [b3[:, _CH - 1, :], jnp.zeros((pad, _T), jnp.float32)],
        axis=0).reshape(_NG, _CH, _T)
    def flat_rowshift(x, s, fill):
        gsh, rsh = s // _CH, s % _CH

        def gshift(y, n):
            if n == 0:
                return y
            return jnp.concatenate(
                [jnp.full((n, _CH, _T), fill, y.dtype), y[:_NG - n]], axis=0)

        if rsh == 0:
            return gshift(x, gsh)
        xr = pltpu.roll(x, rsh, 1)
        return jnp.where(rnp < rsh,
                         gshift(xr, gsh + 1), gshift(xr, gsh))

    s = 1
    while s < _NC:
        cs = _CH * s
        gcs = flat_rowshift(gc, s, 1.0)
        bcs = flat_rowshift(bc, s, 0.0)
        cfill = cnp < cs
        gcs = jnp.where(cfill, 1.0, pltpu.roll(gcs, cs, 2))
        bcs = jnp.where(cfill, 0.0, pltpu.roll(bcs, cs, 2))
        bc = gc * bcs + bc
        gc = gc * gcs
        s *= 2

    pb = bc.reshape(_NG * _CH, _T)
    eb = jnp.concatenate(
        [jnp.zeros((1, _T), jnp.float32), pb[:_NC - 1, :]], axis=0)
    w = jnp.broadcast_to(eb[:, None, :], (_NC, _CH, _T))
    w = pltpu.roll(w, 1, 2, stride=1, stride_axis=1)
    w = jnp.where(cnp <= rnp, 0.0, w)
    b3 = g3 * w + b3
    return b3.reshape(_M, _T)


def _attn_kernel(q_ref, k_ref, v_ref, wo_ref, o_ref, carry_ref):
    rb = pl.program_id(1)
    q = q_ref[0]
    k = k_ref[0]
    a = jax.lax.dot_general(q, k, (((1,), (1,)), ((), ())),
                            preferred_element_type=jnp.float32)

    carry = jnp.where(rb == 0, 0.0, carry_ref[...])
    carry = jnp.concatenate(
        [carry[:, _PRE - 1:], jnp.zeros((1, _PRE - 1), jnp.float32)], axis=1)

    g = jnp.concatenate(
        [jnp.zeros((1, _T), jnp.float32),
         jnp.ones((_PRE - 1, _T), jnp.float32), a], axis=0)
    b = jnp.concatenate(
        [carry, jnp.zeros((_PRE - 1, _T), jnp.float32), a], axis=0)

    b = _scan_stripe(g, b)

    carry_ref[...] = b[_M - 1:_M, :]
    y = b[_PRE:, :]

    rows = rb * _R + jax.lax.broadcasted_iota(jnp.int32, (_R, _T), 0)
    cols = jax.lax.broadcasted_iota(jnp.int32, (_R, _T), 1)
    p = cols.astype(jnp.float32) / (rows.astype(jnp.float32) + 1.0)
    z = jnp.where(cols <= rows, (y + p) * (1.0 / _TAU), -jnp.inf)
    z = z - jnp.max(z, axis=1, keepdims=True)
    e = jnp.exp(z)
    probs = e / jnp.sum(e, axis=1, keepdims=True)

    av = jnp.dot(probs, v_ref[0], preferred_element_type=jnp.float32)
    o_ref[0] = jnp.dot(av, wo_ref[0], preferred_element_type=jnp.float32)


def kernel(x, wq, wk, wv, wo):
    x2 = x.reshape(_T, _DIMS)

    q_sm, k_sm, v_sm = pl.pallas_call(
        _proj_kernel,
        grid=(_T // _RP, _NHEADS),
        in_specs=[
            pl.BlockSpec((_RP, _DIMS), lambda rb, h: (rb, 0)),
            pl.BlockSpec((_HD, _DIMS), lambda rb, h: (h, 0)),
            pl.BlockSpec((_HD, _DIMS), lambda rb, h: (h, 0)),
            pl.BlockSpec((_HD, _DIMS), lambda rb, h: (h, 0)),
        ],
        out_specs=[
            pl.BlockSpec((1, _RP, _HD), lambda rb, h: (h, rb, 0)),
            pl.BlockSpec((1, _RP, _HD), lambda rb, h: (h, rb, 0)),
            pl.BlockSpec((1, _RP, _HD), lambda rb, h: (h, rb, 0)),
        ],
        out_shape=[jax.ShapeDtypeStruct((_NHEADS, _T, _HD), jnp.float32)] * 3,
    )(x2, wq, wk, wv)

    out = pl.pallas_call(
        _attn_kernel,
        grid=(_NHEADS, _T // _R),
        in_specs=[
            pl.BlockSpec((1, _R, _HD), lambda h, rb: (h, rb, 0)),
            pl.BlockSpec((1, _T, _HD), lambda h, rb: (h, 0, 0)),
            pl.BlockSpec((1, _T, _HD), lambda h, rb: (h, 0, 0)),
            pl.BlockSpec((1, _HD, _HD), lambda h, rb: (h, 0, 0)),
        ],
        out_specs=pl.BlockSpec((1, _R, _HD), lambda h, rb: (h, rb, 0)),
        out_shape=jax.ShapeDtypeStruct((_NHEADS, _T, _HD), jnp.float32),
        scratch_shapes=[pltpu.VMEM((1, _T), jnp.float32)],
    )(q_sm, k_sm, v_sm, wo.reshape(_NHEADS, _HD, _HD))

    return out.transpose(1, 0, 2).reshape(_B, _T, _DIMS)


# speedup vs baseline: 135.8543x; 1.3128x over previous
import jax
import jax.numpy as jnp
import numpy as np
from jax.experimental import pallas as pl
from jax.experimental.pallas import tpu as pltpu

_B, _T, _DIMS, _NHEADS = 1, 2048, 768, 12
_HD = _DIMS // _NHEADS
_TAU = 0.1

_RP = 256
_R = 256
_PRE = 8
_M = _R + _PRE
_CH = 8
_NC = _M // _CH


def _proj_kernel(x_ref, wq_ref, wk_ref, wv_ref, q_ref, k_ref, v_ref):
    x = x_ref[...]
    for w_ref, o_ref in ((wq_ref, q_ref), (wk_ref, k_ref), (wv_ref, v_ref)):
        y = jax.lax.dot_general(x, w_ref[...], (((1,), (1,)), ((), ())),
                                preferred_element_type=jnp.float32)
        y = y * (1.0 / _TAU)
        y = y - jnp.max(y, axis=1, keepdims=True)
        e = jnp.exp(y)
        o_ref[0] = e / jnp.sum(e, axis=1, keepdims=True)


def _shift_diag(x, s, fill):
    m, t = x.shape
    x = jnp.concatenate([jnp.full((s, t), fill, x.dtype), x[:m - s, :]], axis=0)
    x = jnp.concatenate([jnp.full((m, s), fill, x.dtype), x[:, :t - s]], axis=1)
    return x


def _shift_cols(x, s, fill):
    sh = x.shape[:-1] + (s,)
    return jnp.concatenate(
        [jnp.full(sh, fill, x.dtype), x[..., :x.shape[-1] - s]], axis=-1)


_NG = (_NC + _CH - 1) // _CH


def _scan_stripe(g, b, width):
    g3 = g.reshape(_NC, _CH, width)
    b3 = b.reshape(_NC, _CH, width)
    rnp = jax.lax.broadcasted_iota(jnp.int32, (1, _CH, width), 1)
    cnp = jax.lax.broadcasted_iota(jnp.int32, (1, _CH, width), 2)

    for s in (1, 2, 4):
        fill = (rnp < s) | (cnp < s)
        gs = jnp.where(fill, 1.0, pltpu.roll(pltpu.roll(g3, s, 1), s, 2))
        bs = jnp.where(fill, 0.0, pltpu.roll(pltpu.roll(b3, s, 1), s, 2))
        b3 = g3 * bs + b3
        g3 = g3 * gs

    pad = _NG * _CH - _NC
    gc = jnp.concatenate(
        [g3[:, _CH - 1, :], jnp.ones((pad, width), jnp.float32)],
        axis=0).reshape(_NG, _CH, width)
    bc = jnp.concatenate(
        [b3[:, _CH - 1, :], jnp.zeros((pad, width), jnp.float32)],
        axis=0).reshape(_NG, _CH, width)
    def flat_rowshift(x, s, fill):
        gsh, rsh = s // _CH, s % _CH

        def gshift(y, n):
            if n == 0:
                return y
            return jnp.concatenate(
                [jnp.full((n, _CH, width), fill, y.dtype), y[:_NG - n]], axis=0)

        if rsh == 0:
            return gshift(x, gsh)
        xr = pltpu.roll(x, rsh, 1)
        return jnp.where(rnp < rsh,
                         gshift(xr, gsh + 1), gshift(xr, gsh))

    s = 1
    while s < _NC:
        cs = _CH * s
        gcs = flat_rowshift(gc, s, 1.0)
        bcs = flat_rowshift(bc, s, 0.0)
        cfill = cnp < cs
        gcs = jnp.where(cfill, 1.0, pltpu.roll(gcs, cs, 2))
        bcs = jnp.where(cfill, 0.0, pltpu.roll(bcs, cs, 2))
        bc = gc * bcs + bc
        gc = gc * gcs
        s *= 2

    pb = bc.reshape(_NG * _CH, width)
    eb = jnp.concatenate(
        [jnp.zeros((1, width), jnp.float32), pb[:_NC - 1, :]], axis=0)
    w = jnp.broadcast_to(eb[:, None, :], (_NC, _CH, width))
    w = pltpu.roll(w, 1, 2, stride=1, stride_axis=1)
    w = jnp.where(cnp <= rnp, 0.0, w)
    b3 = g3 * w + b3
    return b3.reshape(_M, width)


def _make_attn_kernel(w, rb_start):
    def attn(q_ref, k_ref, v_ref, wo_ref, ci_ref, o_ref, co_ref, carry_ref):
        rb = pl.program_id(1)
        q = q_ref[0]
        k = k_ref[0]
        a = jax.lax.dot_general(q, k, (((1,), (1,)), ((), ())),
                                preferred_element_type=jnp.float32)

        cin = ci_ref[0, 0:1, :w]
        carry = jnp.where(rb == 0, cin, carry_ref[...])
        carry = jnp.concatenate(
            [carry[:, _PRE - 1:], jnp.zeros((1, _PRE - 1), jnp.float32)],
            axis=1)

        g = jnp.concatenate(
            [jnp.zeros((1, w), jnp.float32),
             jnp.ones((_PRE - 1, w), jnp.float32), a], axis=0)
        b = jnp.concatenate(
            [carry, jnp.zeros((_PRE - 1, w), jnp.float32), a], axis=0)

        b = _scan_stripe(g, b, w)

        carry_ref[...] = b[_M - 1:_M, :]
        if w == _T:
            co_ref[0] = jnp.broadcast_to(b[_M - 1:_M, :], (_CH, _T))
        else:
            co_ref[0] = jnp.concatenate(
                [jnp.broadcast_to(b[_M - 1:_M, :], (_CH, w)),
                 jnp.zeros((_CH, _T - w), jnp.float32)], axis=1)
        y = b[_PRE:, :]

        rows = ((rb_start * _R + rb * _R)
                + jax.lax.broadcasted_iota(jnp.int32, (_R, w), 0))
        cols = jax.lax.broadcasted_iota(jnp.int32, (_R, w), 1)
        p = cols.astype(jnp.float32) / (rows.astype(jnp.float32) + 1.0)
        z = jnp.where(cols <= rows, (y + p) * (1.0 / _TAU), -jnp.inf)
        z = z - jnp.max(z, axis=1, keepdims=True)
        e = jnp.exp(z)
        probs = e / jnp.sum(e, axis=1, keepdims=True)

        av = jnp.dot(probs, v_ref[0], preferred_element_type=jnp.float32)
        o_ref[0] = jnp.dot(av, wo_ref[0], preferred_element_type=jnp.float32)

    return attn


_STAGES = ((512, 0, 2), (1024, 2, 2), (1536, 4, 2), (2048, 6, 2))


def kernel(x, wq, wk, wv, wo):
    x2 = x.reshape(_T, _DIMS)

    q_sm, k_sm, v_sm = pl.pallas_call(
        _proj_kernel,
        grid=(_T // _RP, _NHEADS),
        in_specs=[
            pl.BlockSpec((_RP, _DIMS), lambda rb, h: (rb, 0)),
            pl.BlockSpec((_HD, _DIMS), lambda rb, h: (h, 0)),
            pl.BlockSpec((_HD, _DIMS), lambda rb, h: (h, 0)),
            pl.BlockSpec((_HD, _DIMS), lambda rb, h: (h, 0)),
        ],
        out_specs=[
            pl.BlockSpec((1, _RP, _HD), lambda rb, h: (h, rb, 0)),
            pl.BlockSpec((1, _RP, _HD), lambda rb, h: (h, rb, 0)),
            pl.BlockSpec((1, _RP, _HD), lambda rb, h: (h, rb, 0)),
        ],
        out_shape=[jax.ShapeDtypeStruct((_NHEADS, _T, _HD), jnp.float32)] * 3,
    )(x2, wq, wk, wv)

    wo3 = wo.reshape(_NHEADS, _HD, _HD)
    ci = jnp.zeros((_NHEADS, _CH, _T), jnp.float32)
    parts = []
    for w, rb0, nrb in _STAGES:
        out_p, co = pl.pallas_call(
            _make_attn_kernel(w, rb0),
            grid=(_NHEADS, nrb),
            in_specs=[
                pl.BlockSpec((1, _R, _HD),
                             lambda h, rb, rb0=rb0: (h, rb0 + rb, 0)),
                pl.BlockSpec((1, w, _HD), lambda h, rb: (h, 0, 0)),
                pl.BlockSpec((1, w, _HD), lambda h, rb: (h, 0, 0)),
                pl.BlockSpec((1, _HD, _HD), lambda h, rb: (h, 0, 0)),
                pl.BlockSpec((1, _CH, _T), lambda h, rb: (h, 0, 0)),
            ],
            out_specs=[
                pl.BlockSpec((1, _R, _HD), lambda h, rb: (h, rb, 0)),
                pl.BlockSpec((1, _CH, _T), lambda h, rb: (h, 0, 0)),
            ],
            out_shape=[
                jax.ShapeDtypeStruct((_NHEADS, nrb * _R, _HD), jnp.float32),
                jax.ShapeDtypeStruct((_NHEADS, _CH, _T), jnp.float32),
            ],
            scratch_shapes=[pltpu.VMEM((1, w), jnp.float32)],
        )(q_sm, k_sm, v_sm, wo3, ci)
        parts.append(out_p)
        ci = co

    out = jnp.concatenate(parts, axis=1)
    return out.transpose(1, 0, 2).reshape(_B, _T, _DIMS)


# speedup vs baseline: 141.8760x; 1.0443x over previous
import jax
import jax.numpy as jnp
import numpy as np
from jax.experimental import pallas as pl
from jax.experimental.pallas import tpu as pltpu

_B, _T, _DIMS, _NHEADS = 1, 2048, 768, 12
_HD = _DIMS // _NHEADS
_TAU = 0.1

_RP = 256
_R = 256
_PRE = 8
_M = _R + _PRE
_CH = 8
_NC = _M // _CH


def _proj_kernel(x_ref, wq_ref, wk_ref, wv_ref, q_ref, k_ref, v_ref):
    x = x_ref[...]
    for w_ref, o_ref in ((wq_ref, q_ref), (wk_ref, k_ref), (wv_ref, v_ref)):
        y = jax.lax.dot_general(x, w_ref[...], (((1,), (1,)), ((), ())),
                                preferred_element_type=jnp.float32)
        y = y * (1.0 / _TAU)
        y = y - jnp.max(y, axis=1, keepdims=True)
        e = jnp.exp(y)
        o_ref[0] = e / jnp.sum(e, axis=1, keepdims=True)


def _shift_diag(x, s, fill):
    m, t = x.shape
    x = jnp.concatenate([jnp.full((s, t), fill, x.dtype), x[:m - s, :]], axis=0)
    x = jnp.concatenate([jnp.full((m, s), fill, x.dtype), x[:, :t - s]], axis=1)
    return x


def _shift_cols(x, s, fill):
    sh = x.shape[:-1] + (s,)
    return jnp.concatenate(
        [jnp.full(sh, fill, x.dtype), x[..., :x.shape[-1] - s]], axis=-1)


_NG = (_NC + _CH - 1) // _CH


def _scan_stripe(g, b, width):
    g3 = g.reshape(_NC, _CH, width)
    b3 = b.reshape(_NC, _CH, width)
    rnp = jax.lax.broadcasted_iota(jnp.int32, (1, _CH, width), 1)
    cnp = jax.lax.broadcasted_iota(jnp.int32, (1, _CH, width), 2)

    for s in (1, 2, 4):
        fill = (rnp < s) | (cnp < s)
        gs = jnp.where(fill, 1.0, pltpu.roll(pltpu.roll(g3, s, 1), s, 2))
        bs = jnp.where(fill, 0.0, pltpu.roll(pltpu.roll(b3, s, 1), s, 2))
        b3 = g3 * bs + b3
        g3 = g3 * gs

    pad = _NG * _CH - _NC
    gc = jnp.concatenate(
        [g3[:, _CH - 1, :], jnp.ones((pad, width), jnp.float32)],
        axis=0).reshape(_NG, _CH, width)
    bc = jnp.concatenate(
        [b3[:, _CH - 1, :], jnp.zeros((pad, width), jnp.float32)],
        axis=0).reshape(_NG, _CH, width)
    def flat_rowshift(x, s, fill):
        gsh, rsh = s // _CH, s % _CH

        def gshift(y, n):
            if n == 0:
                return y
            return jnp.concatenate(
                [jnp.full((n, _CH, width), fill, y.dtype), y[:_NG - n]], axis=0)

        if rsh == 0:
            return gshift(x, gsh)
        xr = pltpu.roll(x, rsh, 1)
        return jnp.where(rnp < rsh,
                         gshift(xr, gsh + 1), gshift(xr, gsh))

    s = 1
    while s < _NC:
        cs = _CH * s
        gcs = flat_rowshift(gc, s, 1.0)
        bcs = flat_rowshift(bc, s, 0.0)
        cfill = cnp < cs
        gcs = jnp.where(cfill, 1.0, pltpu.roll(gcs, cs, 2))
        bcs = jnp.where(cfill, 0.0, pltpu.roll(bcs, cs, 2))
        bc = gc * bcs + bc
        gc = gc * gcs
        s *= 2

    pb = bc.reshape(_NG * _CH, width)
    eb = jnp.concatenate(
        [jnp.zeros((1, width), jnp.float32), pb[:_NC - 1, :]], axis=0)
    w = jnp.broadcast_to(eb[:, None, :], (_NC, _CH, width))
    w = pltpu.roll(w, 1, 2, stride=1, stride_axis=1)
    w = jnp.where(cnp <= rnp, 0.0, w)
    b3 = g3 * w + b3
    return b3.reshape(_M, width)


def _make_attn_kernel(w, rb_start):
    def attn(q_ref, k_ref, v_ref, wo_ref, ci_ref, o_ref, co_ref, carry_ref):
        rb = pl.program_id(1)
        q = q_ref[0]
        k = k_ref[0]
        a = jax.lax.dot_general(q, k, (((1,), (1,)), ((), ())),
                                preferred_element_type=jnp.float32)

        cin = ci_ref[0, 0:1, :w]
        carry = jnp.where(rb == 0, cin, carry_ref[...])
        carry = jnp.concatenate(
            [carry[:, _PRE - 1:], jnp.zeros((1, _PRE - 1), jnp.float32)],
            axis=1)

        g = jnp.concatenate(
            [jnp.zeros((1, w), jnp.float32),
             jnp.ones((_PRE - 1, w), jnp.float32), a], axis=0)
        b = jnp.concatenate(
            [carry, jnp.zeros((_PRE - 1, w), jnp.float32), a], axis=0)

        b = _scan_stripe(g, b, w)

        carry_ref[...] = b[_M - 1:_M, :]
        if w == _T:
            co_ref[0] = jnp.broadcast_to(b[_M - 1:_M, :], (_CH, _T))
        else:
            co_ref[0] = jnp.concatenate(
                [jnp.broadcast_to(b[_M - 1:_M, :], (_CH, w)),
                 jnp.zeros((_CH, _T - w), jnp.float32)], axis=1)
        y = b[_PRE:, :]

        rows = ((rb_start * _R + rb * _R)
                + jax.lax.broadcasted_iota(jnp.int32, (_R, w), 0))
        cols = jax.lax.broadcasted_iota(jnp.int32, (_R, w), 1)
        p = cols.astype(jnp.float32) / (rows.astype(jnp.float32) + 1.0)
        z = jnp.where(cols <= rows, (y + p) * (1.0 / _TAU), -jnp.inf)
        z = z - jnp.max(z, axis=1, keepdims=True)
        e = jnp.exp(z)
        probs = e / jnp.sum(e, axis=1, keepdims=True)

        av = jnp.dot(probs, v_ref[0], preferred_element_type=jnp.float32)
        o_ref[0] = jnp.dot(av, wo_ref[0], preferred_element_type=jnp.float32)

    return attn


_STAGES = tuple((256 * (i + 1), i, 1) for i in range(8))


def kernel(x, wq, wk, wv, wo):
    x2 = x.reshape(_T, _DIMS)

    q_sm, k_sm, v_sm = pl.pallas_call(
        _proj_kernel,
        grid=(_T // _RP, _NHEADS),
        in_specs=[
            pl.BlockSpec((_RP, _DIMS), lambda rb, h: (rb, 0)),
            pl.BlockSpec((_HD, _DIMS), lambda rb, h: (h, 0)),
            pl.BlockSpec((_HD, _DIMS), lambda rb, h: (h, 0)),
            pl.BlockSpec((_HD, _DIMS), lambda rb, h: (h, 0)),
        ],
        out_specs=[
            pl.BlockSpec((1, _RP, _HD), lambda rb, h: (h, rb, 0)),
            pl.BlockSpec((1, _RP, _HD), lambda rb, h: (h, rb, 0)),
            pl.BlockSpec((1, _RP, _HD), lambda rb, h: (h, rb, 0)),
        ],
        out_shape=[jax.ShapeDtypeStruct((_NHEADS, _T, _HD), jnp.float32)] * 3,
    )(x2, wq, wk, wv)

    wo3 = wo.reshape(_NHEADS, _HD, _HD)
    ci = jnp.zeros((_NHEADS, _CH, _T), jnp.float32)
    parts = []
    for w, rb0, nrb in _STAGES:
        out_p, co = pl.pallas_call(
            _make_attn_kernel(w, rb0),
            grid=(_NHEADS, nrb),
            in_specs=[
                pl.BlockSpec((1, _R, _HD),
                             lambda h, rb, rb0=rb0: (h, rb0 + rb, 0)),
                pl.BlockSpec((1, w, _HD), lambda h, rb: (h, 0, 0)),
                pl.BlockSpec((1, w, _HD), lambda h, rb: (h, 0, 0)),
                pl.BlockSpec((1, _HD, _HD), lambda h, rb: (h, 0, 0)),
                pl.BlockSpec((1, _CH, _T), lambda h, rb: (h, 0, 0)),
            ],
            out_specs=[
                pl.BlockSpec((1, _R, _HD), lambda h, rb: (h, rb, 0)),
                pl.BlockSpec((1, _CH, _T), lambda h, rb: (h, 0, 0)),
            ],
            out_shape=[
                jax.ShapeDtypeStruct((_NHEADS, nrb * _R, _HD), jnp.float32),
                jax.ShapeDtypeStruct((_NHEADS, _CH, _T), jnp.float32),
            ],
            scratch_shapes=[pltpu.VMEM((1, w), jnp.float32)],
        )(q_sm, k_sm, v_sm, wo3, ci)
        parts.append(out_p)
        ci = co

    out = jnp.concatenate(parts, axis=1)
    return out.transpose(1, 0, 2).reshape(_B, _T, _DIMS)


# speedup vs baseline: 152.7255x; 1.0765x over previous
import jax
import jax.numpy as jnp
import numpy as np
from jax.experimental import pallas as pl
from jax.experimental.pallas import tpu as pltpu

_B, _T, _DIMS, _NHEADS = 1, 2048, 768, 12
_HD = _DIMS // _NHEADS
_TAU = 0.1

_RP = 256
_R = 256
_PRE = 8
_M = _R + _PRE
_CH = 8
_NC = _M // _CH


def _proj_kernel(x_ref, wq_ref, wk_ref, wv_ref, q_ref, k_ref, v_ref):
    x = x_ref[...]
    for w_ref, o_ref in ((wq_ref, q_ref), (wk_ref, k_ref), (wv_ref, v_ref)):
        y = jax.lax.dot_general(x, w_ref[...], (((1,), (1,)), ((), ())),
                                preferred_element_type=jnp.float32)
        y = y * (1.0 / _TAU)
        y = y - jnp.max(y, axis=1, keepdims=True)
        e = jnp.exp(y)
        o_ref[0] = e / jnp.sum(e, axis=1, keepdims=True)


def _shift_diag(x, s, fill):
    m, t = x.shape
    x = jnp.concatenate([jnp.full((s, t), fill, x.dtype), x[:m - s, :]], axis=0)
    x = jnp.concatenate([jnp.full((m, s), fill, x.dtype), x[:, :t - s]], axis=1)
    return x


def _shift_cols(x, s, fill):
    sh = x.shape[:-1] + (s,)
    return jnp.concatenate(
        [jnp.full(sh, fill, x.dtype), x[..., :x.shape[-1] - s]], axis=-1)


_NG = (_NC + _CH - 1) // _CH


def _scan_stripe(g, b, width):
    g3 = g.reshape(_NC, _CH, width)
    b3 = b.reshape(_NC, _CH, width)
    rnp = jax.lax.broadcasted_iota(jnp.int32, (1, _CH, width), 1)
    cnp = jax.lax.broadcasted_iota(jnp.int32, (1, _CH, width), 2)

    for s in (1, 2, 4):
        fill = (rnp < s) | (cnp < s)
        gs = jnp.where(fill, 1.0, pltpu.roll(pltpu.roll(g3, s, 1), s, 2))
        bs = jnp.where(fill, 0.0, pltpu.roll(pltpu.roll(b3, s, 1), s, 2))
        b3 = g3 * bs + b3
        g3 = g3 * gs

    pad = _NG * _CH - _NC
    gc = jnp.concatenate(
        [g3[:, _CH - 1, :], jnp.ones((pad, width), jnp.float32)],
        axis=0).reshape(_NG, _CH, width)
    bc = jnp.concatenate(
        [b3[:, _CH - 1, :], jnp.zeros((pad, width), jnp.float32)],
        axis=0).reshape(_NG, _CH, width)
    def flat_rowshift(x, s, fill):
        gsh, rsh = s // _CH, s % _CH

        def gshift(y, n):
            if n == 0:
                return y
            return jnp.concatenate(
                [jnp.full((n, _CH, width), fill, y.dtype), y[:_NG - n]], axis=0)

        if rsh == 0:
            return gshift(x, gsh)
        xr = pltpu.roll(x, rsh, 1)
        return jnp.where(rnp < rsh,
                         gshift(xr, gsh + 1), gshift(xr, gsh))

    s = 1
    while s < _NC:
        cs = _CH * s
        gcs = flat_rowshift(gc, s, 1.0)
        bcs = flat_rowshift(bc, s, 0.0)
        cfill = cnp < cs
        gcs = jnp.where(cfill, 1.0, pltpu.roll(gcs, cs, 2))
        bcs = jnp.where(cfill, 0.0, pltpu.roll(bcs, cs, 2))
        bc = gc * bcs + bc
        gc = gc * gcs
        s *= 2

    pb = bc.reshape(_NG * _CH, width)
    eb = jnp.concatenate(
        [jnp.zeros((1, width), jnp.float32), pb[:_NC - 1, :]], axis=0)
    w = jnp.broadcast_to(eb[:, None, :], (_NC, _CH, width))
    w = pltpu.roll(w, 1, 2, stride=1, stride_axis=1)
    w = jnp.where(cnp <= rnp, 0.0, w)
    b3 = g3 * w + b3
    return b3.reshape(_M, width)


def _make_attn_kernel(w, rb0):
    def attn(q_ref, k_ref, v_ref, wo_ref, ci_ref, o_ref, co_ref):
        h = pl.program_id(0)
        q = q_ref[0]
        k = k_ref[0]
        a = jax.lax.dot_general(q, k, (((1,), (1,)), ((), ())),
                                preferred_element_type=jnp.float32)

        carry = ci_ref[0, 0:1, :w]
        carry = jnp.concatenate(
            [carry[:, _PRE - 1:], jnp.zeros((1, _PRE - 1), jnp.float32)],
            axis=1)

        g = jnp.concatenate(
            [jnp.zeros((1, w), jnp.float32),
             jnp.ones((_PRE - 1, w), jnp.float32), a], axis=0)
        b = jnp.concatenate(
            [carry, jnp.zeros((_PRE - 1, w), jnp.float32), a], axis=0)

        b = _scan_stripe(g, b, w)

        if w == _T:
            co_ref[0] = jnp.broadcast_to(b[_M - 1:_M, :], (_CH, _T))
        else:
            co_ref[0] = jnp.concatenate(
                [jnp.broadcast_to(b[_M - 1:_M, :], (_CH, w)),
                 jnp.zeros((_CH, _T - w), jnp.float32)], axis=1)
        y = b[_PRE:, :]

        rows = rb0 * _R + jax.lax.broadcasted_iota(jnp.int32, (_R, w), 0)
        cols = jax.lax.broadcasted_iota(jnp.int32, (_R, w), 1)
        colsf = cols.astype(jnp.float32)
        rinv = (1.0 / _TAU) / (
            rb0 * _R + 1.0
            + jax.lax.broadcasted_iota(jnp.int32, (_R, 1), 0).astype(jnp.float32))
        z = jnp.where(cols <= rows, y * (1.0 / _TAU) + colsf * rinv, -jnp.inf)
        z = z - jnp.max(z, axis=1, keepdims=True)
        e = jnp.exp(z)
        probs = e / jnp.sum(e, axis=1, keepdims=True)

        av = jnp.dot(probs, v_ref[0], preferred_element_type=jnp.float32)
        res = jnp.dot(av, wo_ref[0], preferred_element_type=jnp.float32)

        zpad = jnp.zeros((_R, _HD), jnp.float32)
        o_ref[...] = jnp.where(
            h % 2 == 0,
            jnp.concatenate([res, zpad], axis=1),
            o_ref[...] + jnp.concatenate([zpad, res], axis=1))

    return attn


def kernel(x, wq, wk, wv, wo):
    x2 = x.reshape(_T, _DIMS)

    q_sm, k_sm, v_sm = pl.pallas_call(
        _proj_kernel,
        grid=(_T // _RP, _NHEADS),
        in_specs=[
            pl.BlockSpec((_RP, _DIMS), lambda rb, h: (rb, 0)),
            pl.BlockSpec((_HD, _DIMS), lambda rb, h: (h, 0)),
            pl.BlockSpec((_HD, _DIMS), lambda rb, h: (h, 0)),
            pl.BlockSpec((_HD, _DIMS), lambda rb, h: (h, 0)),
        ],
        out_specs=[
            pl.BlockSpec((1, _RP, _HD), lambda rb, h: (h, rb, 0)),
            pl.BlockSpec((1, _RP, _HD), lambda rb, h: (h, rb, 0)),
            pl.BlockSpec((1, _RP, _HD), lambda rb, h: (h, rb, 0)),
        ],
        out_shape=[jax.ShapeDtypeStruct((_NHEADS, _T, _HD), jnp.float32)] * 3,
    )(x2, wq, wk, wv)

    wo3 = wo.reshape(_NHEADS, _HD, _HD)
    ci = jnp.zeros((_NHEADS, _CH, _T), jnp.float32)
    parts = []
    for rb0 in range(_T // _R):
        w = _R * (rb0 + 1)
        out_p, co = pl.pallas_call(
            _make_attn_kernel(w, rb0),
            grid=(_NHEADS,),
            in_specs=[
                pl.BlockSpec((1, _R, _HD), lambda h, rb0=rb0: (h, rb0, 0)),
                pl.BlockSpec((1, w, _HD), lambda h: (h, 0, 0)),
                pl.BlockSpec((1, w, _HD), lambda h: (h, 0, 0)),
                pl.BlockSpec((1, _HD, _HD), lambda h: (h, 0, 0)),
                pl.BlockSpec((1, _CH, _T), lambda h: (h, 0, 0)),
            ],
            out_specs=[
                pl.BlockSpec((_R, 2 * _HD), lambda h: (0, h // 2)),
                pl.BlockSpec((1, _CH, _T), lambda h: (h, 0, 0)),
            ],
            out_shape=[
                jax.ShapeDtypeStruct((_R, _DIMS), jnp.float32),
                jax.ShapeDtypeStruct((_NHEADS, _CH, _T), jnp.float32),
            ],
        )(q_sm, k_sm, v_sm, wo3, ci)
        parts.append(out_p)
        ci = co

    return jnp.concatenate(parts, axis=0).reshape(_B, _T, _DIMS)


# speedup vs baseline: 163.6697x; 1.0717x over previous
import jax
import jax.numpy as jnp
import numpy as np
from jax.experimental import pallas as pl
from jax.experimental.pallas import tpu as pltpu

_B, _T, _DIMS, _NHEADS = 1, 2048, 768, 12
_HD = _DIMS // _NHEADS
_TAU = 0.1

_RP = 256
_R = 256
_PRE = 8
_M = _R + _PRE
_CH = 8
_NC = _M // _CH


def _proj_kernel(x_ref, wq_ref, wk_ref, wv_ref, q_ref, k_ref, v_ref):
    x = x_ref[...]
    for w_ref, o_ref in ((wq_ref, q_ref), (wk_ref, k_ref), (wv_ref, v_ref)):
        y = jax.lax.dot_general(x, w_ref[...], (((1,), (1,)), ((), ())),
                                preferred_element_type=jnp.float32)
        y = y * (1.0 / _TAU)
        y = y - jnp.max(y, axis=1, keepdims=True)
        e = jnp.exp(y)
        o_ref[0] = e / jnp.sum(e, axis=1, keepdims=True)


def _shift_diag(x, s, fill):
    m, t = x.shape
    x = jnp.concatenate([jnp.full((s, t), fill, x.dtype), x[:m - s, :]], axis=0)
    x = jnp.concatenate([jnp.full((m, s), fill, x.dtype), x[:, :t - s]], axis=1)
    return x


def _shift_cols(x, s, fill):
    sh = x.shape[:-1] + (s,)
    return jnp.concatenate(
        [jnp.full(sh, fill, x.dtype), x[..., :x.shape[-1] - s]], axis=-1)


_NG = (_NC + _CH - 1) // _CH


def _scan_stripe(g, b, hb, width):
    g3 = g.reshape(hb * _NC, _CH, width)
    b3 = b.reshape(hb * _NC, _CH, width)
    rnp = jax.lax.broadcasted_iota(jnp.int32, (1, _CH, width), 1)
    cnp = jax.lax.broadcasted_iota(jnp.int32, (1, _CH, width), 2)

    for s in (1, 2, 4):
        fill = (rnp < s) | (cnp < s)
        gs = jnp.where(fill, 1.0, pltpu.roll(pltpu.roll(g3, s, 1), s, 2))
        bs = jnp.where(fill, 0.0, pltpu.roll(pltpu.roll(b3, s, 1), s, 2))
        b3 = g3 * bs + b3
        g3 = g3 * gs

    pad = _NG * _CH - _NC
    lastrow = g3.reshape(hb, _NC, _CH, width)[:, :, _CH - 1, :]
    gc = jnp.concatenate(
        [lastrow, jnp.ones((hb, pad, width), jnp.float32)],
        axis=1).reshape(hb, _NG, _CH, width)
    lastrow = b3.reshape(hb, _NC, _CH, width)[:, :, _CH - 1, :]
    bc = jnp.concatenate(
        [lastrow, jnp.zeros((hb, pad, width), jnp.float32)],
        axis=1).reshape(hb, _NG, _CH, width)
    rnp4 = rnp[None]
    cnp4 = cnp[None]

    def flat_rowshift(x, s, fill):
        gsh, rsh = s // _CH, s % _CH

        def gshift(y, n):
            if n == 0:
                return y
            return jnp.concatenate(
                [jnp.full((hb, n, _CH, width), fill, y.dtype),
                 y[:, :_NG - n]], axis=1)

        if rsh == 0:
            return gshift(x, gsh)
        xr = pltpu.roll(x, rsh, 2)
        return jnp.where(rnp4 < rsh, gshift(xr, gsh + 1), gshift(xr, gsh))

    s = 1
    while s < _NC:
        cs = _CH * s
        gcs = flat_rowshift(gc, s, 1.0)
        bcs = flat_rowshift(bc, s, 0.0)
        cfill = cnp4 < cs
        gcs = jnp.where(cfill, 1.0, pltpu.roll(gcs, cs, 3))
        bcs = jnp.where(cfill, 0.0, pltpu.roll(bcs, cs, 3))
        bc = gc * bcs + bc
        gc = gc * gcs
        s *= 2

    pb = bc.reshape(hb, _NG * _CH, width)
    eb = jnp.concatenate(
        [jnp.zeros((hb, 1, width), jnp.float32), pb[:, :_NC - 1, :]], axis=1)
    w = jnp.broadcast_to(eb[:, :, None, :], (hb, _NC, _CH, width))
    w = w.reshape(hb * _NC, _CH, width)
    w = pltpu.roll(w, 1, 2, stride=1, stride_axis=1)
    w = jnp.where(cnp <= rnp, 0.0, w)
    b3 = g3 * w + b3
    return b3.reshape(hb, _M, width)


def _make_attn_kernel(w, rb0, hb):
    def attn(q_ref, k_ref, v_ref, wo_ref, ci_ref, o_ref, co_ref):
        a = [jax.lax.dot_general(q_ref[i], k_ref[i], (((1,), (1,)), ((), ())),
                                 preferred_element_type=jnp.float32)
             for i in range(hb)]
        a = jnp.concatenate([x[None] for x in a], axis=0)

        carry = ci_ref[:, 0:1, :w]
        carry = jnp.concatenate(
            [carry[:, :, _PRE - 1:], jnp.zeros((hb, 1, _PRE - 1), jnp.float32)],
            axis=2)

        g = jnp.concatenate(
            [jnp.zeros((hb, 1, w), jnp.float32),
             jnp.ones((hb, _PRE - 1, w), jnp.float32), a], axis=1)
        b = jnp.concatenate(
            [carry, jnp.zeros((hb, _PRE - 1, w), jnp.float32), a], axis=1)

        b = _scan_stripe(g, b, hb, w)

        last = b[:, _M - 1:_M, :]
        if w == _T:
            co_ref[...] = jnp.broadcast_to(last, (hb, _CH, _T))
        else:
            co_ref[...] = jnp.concatenate(
                [jnp.broadcast_to(last, (hb, _CH, w)),
                 jnp.zeros((hb, _CH, _T - w), jnp.float32)], axis=2)
        y = b[:, _PRE:, :]

        rows = rb0 * _R + jax.lax.broadcasted_iota(jnp.int32, (1, _R, w), 1)
        cols = jax.lax.broadcasted_iota(jnp.int32, (1, _R, w), 2)
        colsf = cols.astype(jnp.float32)
        rinv = (1.0 / _TAU) / (
            rb0 * _R + 1.0
            + jax.lax.broadcasted_iota(jnp.int32, (1, _R, 1), 1)
            .astype(jnp.float32))
        z = jnp.where(cols <= rows, y * (1.0 / _TAU) + colsf * rinv, -jnp.inf)
        z = z - jnp.max(z, axis=2, keepdims=True)
        e = jnp.exp(z)
        probs = e / jnp.sum(e, axis=2, keepdims=True)

        res = [jnp.dot(jnp.dot(probs[i], v_ref[i],
                               preferred_element_type=jnp.float32),
                       wo_ref[i], preferred_element_type=jnp.float32)
               for i in range(hb)]
        o_ref[...] = jnp.concatenate(res, axis=1)

    return attn


def kernel(x, wq, wk, wv, wo):
    x2 = x.reshape(_T, _DIMS)

    q_sm, k_sm, v_sm = pl.pallas_call(
        _proj_kernel,
        grid=(_T // _RP, _NHEADS),
        in_specs=[
            pl.BlockSpec((_RP, _DIMS), lambda rb, h: (rb, 0)),
            pl.BlockSpec((_HD, _DIMS), lambda rb, h: (h, 0)),
            pl.BlockSpec((_HD, _DIMS), lambda rb, h: (h, 0)),
            pl.BlockSpec((_HD, _DIMS), lambda rb, h: (h, 0)),
        ],
        out_specs=[
            pl.BlockSpec((1, _RP, _HD), lambda rb, h: (h, rb, 0)),
            pl.BlockSpec((1, _RP, _HD), lambda rb, h: (h, rb, 0)),
            pl.BlockSpec((1, _RP, _HD), lambda rb, h: (h, rb, 0)),
        ],
        out_shape=[jax.ShapeDtypeStruct((_NHEADS, _T, _HD), jnp.float32)] * 3,
    )(x2, wq, wk, wv)

    wo3 = wo.reshape(_NHEADS, _HD, _HD)
    ci = jnp.zeros((_NHEADS, _CH, _T), jnp.float32)
    parts = []
    for rb0 in range(_T // _R):
        w = _R * (rb0 + 1)
        hb = 6 if w <= 256 else (4 if w <= 1024 else 2)
        out_p, co = pl.pallas_call(
            _make_attn_kernel(w, rb0, hb),
            grid=(_NHEADS // hb,),
            in_specs=[
                pl.BlockSpec((hb, _R, _HD), lambda i, rb0=rb0: (i, rb0, 0)),
                pl.BlockSpec((hb, w, _HD), lambda i: (i, 0, 0)),
                pl.BlockSpec((hb, w, _HD), lambda i: (i, 0, 0)),
                pl.BlockSpec((hb, _HD, _HD), lambda i: (i, 0, 0)),
                pl.BlockSpec((hb, _CH, _T), lambda i: (i, 0, 0)),
            ],
            out_specs=[
                pl.BlockSpec((_R, hb * _HD), lambda i: (0, i)),
                pl.BlockSpec((hb, _CH, _T), lambda i: (i, 0, 0)),
            ],
            out_shape=[
                jax.ShapeDtypeStruct((_R, _DIMS), jnp.float32),
                jax.ShapeDtypeStruct((_NHEADS, _CH, _T), jnp.float32),
            ],
        )(q_sm, k_sm, v_sm, wo3, ci)
        parts.append(out_p)
        ci = co

    return jnp.concatenate(parts, axis=0).reshape(_B, _T, _DIMS)


# speedup vs baseline: 167.9050x; 1.0259x over previous
import jax
import jax.numpy as jnp
import numpy as np
from jax.experimental import pallas as pl
from jax.experimental.pallas import tpu as pltpu

_B, _T, _DIMS, _NHEADS = 1, 2048, 768, 12
_HD = _DIMS // _NHEADS
_TAU = 0.1

_RP = 256
_R = 256
_PRE = 8
_M = _R + _PRE
_CH = 8
_NC = _M // _CH


def _proj_kernel(x_ref, w_ref, q_ref, k_ref, v_ref):
    x = x_ref[...]
    y = jax.lax.dot_general(x, w_ref[...], (((1,), (1,)), ((), ())),
                            preferred_element_type=jnp.float32)
    for t, o_ref in enumerate((q_ref, k_ref, v_ref)):
        for h in range(_NHEADS):
            sl = y[:, (t * _NHEADS + h) * _HD:(t * _NHEADS + h + 1) * _HD]
            sl = sl * (1.0 / _TAU)
            sl = sl - jnp.max(sl, axis=1, keepdims=True)
            e = jnp.exp(sl)
            o_ref[h] = e / jnp.sum(e, axis=1, keepdims=True)


def _shift_diag(x, s, fill):
    m, t = x.shape
    x = jnp.concatenate([jnp.full((s, t), fill, x.dtype), x[:m - s, :]], axis=0)
    x = jnp.concatenate([jnp.full((m, s), fill, x.dtype), x[:, :t - s]], axis=1)
    return x


def _shift_cols(x, s, fill):
    sh = x.shape[:-1] + (s,)
    return jnp.concatenate(
        [jnp.full(sh, fill, x.dtype), x[..., :x.shape[-1] - s]], axis=-1)


_NG = (_NC + _CH - 1) // _CH


def _scan_stripe(g, b, hb, width):
    g3 = g.reshape(hb * _NC, _CH, width)
    b3 = b.reshape(hb * _NC, _CH, width)
    rnp = jax.lax.broadcasted_iota(jnp.int32, (1, _CH, width), 1)
    cnp = jax.lax.broadcasted_iota(jnp.int32, (1, _CH, width), 2)

    for s in (1, 2, 4):
        fill = (rnp < s) | (cnp < s)
        gs = jnp.where(fill, 1.0, pltpu.roll(pltpu.roll(g3, s, 1), s, 2))
        bs = jnp.where(fill, 0.0, pltpu.roll(pltpu.roll(b3, s, 1), s, 2))
        b3 = g3 * bs + b3
        g3 = g3 * gs

    pad = _NG * _CH - _NC
    lastrow = g3.reshape(hb, _NC, _CH, width)[:, :, _CH - 1, :]
    gc = jnp.concatenate(
        [lastrow, jnp.ones((hb, pad, width), jnp.float32)],
        axis=1).reshape(hb, _NG, _CH, width)
    lastrow = b3.reshape(hb, _NC, _CH, width)[:, :, _CH - 1, :]
    bc = jnp.concatenate(
        [lastrow, jnp.zeros((hb, pad, width), jnp.float32)],
        axis=1).reshape(hb, _NG, _CH, width)
    rnp4 = rnp[None]
    cnp4 = cnp[None]

    def flat_rowshift(x, s, fill):
        gsh, rsh = s // _CH, s % _CH

        def gshift(y, n):
            if n == 0:
                return y
            return jnp.concatenate(
                [jnp.full((hb, n, _CH, width), fill, y.dtype),
                 y[:, :_NG - n]], axis=1)

        if rsh == 0:
            return gshift(x, gsh)
        xr = pltpu.roll(x, rsh, 2)
        return jnp.where(rnp4 < rsh, gshift(xr, gsh + 1), gshift(xr, gsh))

    s = 1
    while s < _NC:
        cs = _CH * s
        gcs = flat_rowshift(gc, s, 1.0)
        bcs = flat_rowshift(bc, s, 0.0)
        cfill = cnp4 < cs
        gcs = jnp.where(cfill, 1.0, pltpu.roll(gcs, cs, 3))
        bcs = jnp.where(cfill, 0.0, pltpu.roll(bcs, cs, 3))
        bc = gc * bcs + bc
        gc = gc * gcs
        s *= 2

    pb = bc.reshape(hb, _NG * _CH, width)
    eb = jnp.concatenate(
        [jnp.zeros((hb, 1, width), jnp.float32), pb[:, :_NC - 1, :]], axis=1)
    w = jnp.broadcast_to(eb[:, :, None, :], (hb, _NC, _CH, width))
    w = w.reshape(hb * _NC, _CH, width)
    w = pltpu.roll(w, 1, 2, stride=1, stride_axis=1)
    w = jnp.where(cnp <= rnp, 0.0, w)
    b3 = g3 * w + b3
    return b3.reshape(hb, _M, width)


def _make_attn_kernel(w, rb0, hb):
    def attn(q_ref, k_ref, v_ref, wo_ref, ci_ref, o_ref, co_ref):
        a = [jax.lax.dot_general(q_ref[i], k_ref[i], (((1,), (1,)), ((), ())),
                                 preferred_element_type=jnp.float32)
             for i in range(hb)]
        a = jnp.concatenate([x[None] for x in a], axis=0)

        carry = ci_ref[:, 0:1, :w]
        carry = jnp.concatenate(
            [carry[:, :, _PRE - 1:], jnp.zeros((hb, 1, _PRE - 1), jnp.float32)],
            axis=2)

        g = jnp.concatenate(
            [jnp.zeros((hb, 1, w), jnp.float32),
             jnp.ones((hb, _PRE - 1, w), jnp.float32), a], axis=1)
        b = jnp.concatenate(
            [carry, jnp.zeros((hb, _PRE - 1, w), jnp.float32), a], axis=1)

        b = _scan_stripe(g, b, hb, w)

        last = b[:, _M - 1:_M, :]
        if w == _T:
            co_ref[...] = jnp.broadcast_to(last, (hb, _CH, _T))
        else:
            co_ref[...] = jnp.concatenate(
                [jnp.broadcast_to(last, (hb, _CH, w)),
                 jnp.zeros((hb, _CH, _T - w), jnp.float32)], axis=2)
        y = b[:, _PRE:, :]

        rows = rb0 * _R + jax.lax.broadcasted_iota(jnp.int32, (1, _R, w), 1)
        cols = jax.lax.broadcasted_iota(jnp.int32, (1, _R, w), 2)
        colsf = cols.astype(jnp.float32)
        rinv = (1.0 / _TAU) / (
            rb0 * _R + 1.0
            + jax.lax.broadcasted_iota(jnp.int32, (1, _R, 1), 1)
            .astype(jnp.float32))
        z = jnp.where(cols <= rows, y * (1.0 / _TAU) + colsf * rinv, -jnp.inf)
        z = z - jnp.max(z, axis=2, keepdims=True)
        e = jnp.exp(z)
        probs = e / jnp.sum(e, axis=2, keepdims=True)

        res = [jnp.dot(jnp.dot(probs[i], v_ref[i],
                               preferred_element_type=jnp.float32),
                       wo_ref[i], preferred_element_type=jnp.float32)
               for i in range(hb)]
        o_ref[...] = jnp.concatenate(res, axis=1)

    return attn


def kernel(x, wq, wk, wv, wo):
    x2 = x.reshape(_T, _DIMS)

    wcat = jnp.concatenate([wq, wk, wv], axis=0)
    q_sm, k_sm, v_sm = pl.pallas_call(
        _proj_kernel,
        grid=(_T // _RP,),
        in_specs=[
            pl.BlockSpec((_RP, _DIMS), lambda rb: (rb, 0)),
            pl.BlockSpec((3 * _DIMS, _DIMS), lambda rb: (0, 0)),
        ],
        out_specs=[
            pl.BlockSpec((_NHEADS, _RP, _HD), lambda rb: (0, rb, 0)),
            pl.BlockSpec((_NHEADS, _RP, _HD), lambda rb: (0, rb, 0)),
            pl.BlockSpec((_NHEADS, _RP, _HD), lambda rb: (0, rb, 0)),
        ],
        out_shape=[jax.ShapeDtypeStruct((_NHEADS, _T, _HD), jnp.float32)] * 3,
    )(x2, wcat)

    wo3 = wo.reshape(_NHEADS, _HD, _HD)
    ci = jnp.zeros((_NHEADS, _CH, _T), jnp.float32)
    parts = []
    for rb0 in range(_T // _R):
        w = _R * (rb0 + 1)
        hb = 6 if w <= 256 else (4 if w <= 1024 else 2)
        out_p, co = pl.pallas_call(
            _make_attn_kernel(w, rb0, hb),
            grid=(_NHEADS // hb,),
            in_specs=[
                pl.BlockSpec((hb, _R, _HD), lambda i, rb0=rb0: (i, rb0, 0)),
                pl.BlockSpec((hb, w, _HD), lambda i: (i, 0, 0)),
                pl.BlockSpec((hb, w, _HD), lambda i: (i, 0, 0)),
                pl.BlockSpec((hb, _HD, _HD), lambda i: (i, 0, 0)),
                pl.BlockSpec((hb, _CH, _T), lambda i: (i, 0, 0)),
            ],
            out_specs=[
                pl.BlockSpec((_R, hb * _HD), lambda i: (0, i)),
                pl.BlockSpec((hb, _CH, _T), lambda i: (i, 0, 0)),
            ],
            out_shape=[
                jax.ShapeDtypeStruct((_R, _DIMS), jnp.float32),
                jax.ShapeDtypeStruct((_NHEADS, _CH, _T), jnp.float32),
            ],
        )(q_sm, k_sm, v_sm, wo3, ci)
        parts.append(out_p)
        ci = co

    return jnp.concatenate(parts, axis=0).reshape(_B, _T, _DIMS)
